# baseline (speedup 1.0000x reference)
import functools

import jax
import jax.numpy as jnp
from jax import lax
from jax.experimental import pallas as pl
from jax.experimental.pallas import tpu as pltpu
from jax.experimental.pallas import tpu_sc as plsc

_NC = 2
_NS = 16
_NW = _NC * _NS

_CH = 3328


def _gather_sc(flat_idx, flat_table):
    R = flat_idx.shape[0]
    E = flat_table.shape[1]
    per_w = R // _NW
    n_chunks = per_w // _CH

    mesh = plsc.VectorSubcoreMesh(core_axis_name="c", subcore_axis_name="s")

    @functools.partial(
        pl.kernel,
        mesh=mesh,
        compiler_params=pltpu.CompilerParams(use_tc_tiling_on_sc=False),
        out_type=jax.ShapeDtypeStruct((R, E), jnp.bfloat16),
        scratch_types=[
            pltpu.VMEM((_CH,), jnp.int32),
            pltpu.VMEM((_CH, E), jnp.bfloat16),
            pltpu.SemaphoreType.DMA,
        ],
    )
    def k(idx_hbm, tab_hbm, out_hbm, ids_v, rows_v, sem):
        wid = lax.axis_index("s") * _NC + lax.axis_index("c")
        r0 = wid * per_w

        def chunk(j, _):
            rj = r0 + j * _CH
            pltpu.sync_copy(idx_hbm.at[pl.ds(rj, _CH)], ids_v)
            pltpu.async_copy(tab_hbm.at[ids_v], rows_v, sem).wait()
            pltpu.sync_copy(rows_v, out_hbm.at[pl.ds(rj, _CH)])
            return 0

        lax.fori_loop(0, n_chunks, chunk, 0)

    return k(flat_idx, flat_table)


def _project_tc(feats, w, b):
    B, K = feats.shape
    H = w.shape[1]
    blk = 2048

    def mm(f_ref, w_ref, b_ref, o_ref):
        o_ref[...] = (
            jnp.dot(f_ref[...], w_ref[...], preferred_element_type=jnp.float32)
            + b_ref[...]
        )

    return pl.pallas_call(
        mm,
        grid=(B // blk,),
        in_specs=[
            pl.BlockSpec((blk, K), lambda i: (i, 0)),
            pl.BlockSpec((K, H), lambda i: (0, 0)),
            pl.BlockSpec((1, H), lambda i: (0, 0)),
        ],
        out_specs=pl.BlockSpec((blk, H), lambda i: (i, 0)),
        out_shape=jax.ShapeDtypeStruct((B, H), jnp.float32),
    )(feats, w, b.reshape(1, H))


def kernel(indices, tables, proj_w, proj_b):
    F, B = indices.shape
    V, E = tables.shape[1], tables.shape[2]
    offs = (jnp.arange(F, dtype=jnp.int32) * V)[:, None]
    flat_idx = (indices + offs).T.reshape(F * B)
    flat_table = tables.astype(jnp.bfloat16).reshape(F * V, E)
    feats = _gather_sc(flat_idx, flat_table).reshape(B, F * E)
    return _project_tc(feats, proj_w.astype(jnp.bfloat16), proj_b)


# speedup vs baseline: 1.2032x; 1.2032x over previous
import functools

import jax
import jax.numpy as jnp
from jax import lax
from jax.experimental import pallas as pl
from jax.experimental.pallas import tpu as pltpu
from jax.experimental.pallas import tpu_sc as plsc

_NC = 2
_NS = 16
_NW = _NC * _NS

_CH = 3328


def _gather_sc(flat_idx, flat_table):
    R = flat_idx.shape[0]
    E = flat_table.shape[1]
    per_w = R // _NW
    n_chunks = per_w // _CH

    mesh = plsc.VectorSubcoreMesh(core_axis_name="c", subcore_axis_name="s")

    @functools.partial(
        pl.kernel,
        mesh=mesh,
        compiler_params=pltpu.CompilerParams(use_tc_tiling_on_sc=False),
        out_type=jax.ShapeDtypeStruct((R, E), jnp.float32),
        scratch_types=[
            pltpu.VMEM((_CH,), jnp.int32),
            pltpu.VMEM((_CH, E), jnp.float32),
            pltpu.SemaphoreType.DMA,
        ],
    )
    def k(idx_hbm, tab_hbm, out_hbm, ids_v, rows_v, sem):
        wid = lax.axis_index("s") * _NC + lax.axis_index("c")
        r0 = wid * per_w

        def chunk(j, _):
            rj = r0 + j * _CH
            pltpu.sync_copy(idx_hbm.at[pl.ds(rj, _CH)], ids_v)
            pltpu.async_copy(tab_hbm.at[ids_v], rows_v, sem).wait()
            pltpu.sync_copy(rows_v, out_hbm.at[pl.ds(rj, _CH)])
            return 0

        lax.fori_loop(0, n_chunks, chunk, 0)

    return k(flat_idx, flat_table)


def _project_tc(feats, w, b):
    B, K = feats.shape
    H = w.shape[1]
    blk = 2048

    def mm(f_ref, w_ref, b_ref, o_ref):
        o_ref[...] = (
            jnp.dot(f_ref[...], w_ref[...], preferred_element_type=jnp.float32)
            + b_ref[...]
        )

    return pl.pallas_call(
        mm,
        grid=(B // blk,),
        in_specs=[
            pl.BlockSpec((blk, K), lambda i: (i, 0)),
            pl.BlockSpec((K, H), lambda i: (0, 0)),
            pl.BlockSpec((1, H), lambda i: (0, 0)),
        ],
        out_specs=pl.BlockSpec((blk, H), lambda i: (i, 0)),
        out_shape=jax.ShapeDtypeStruct((B, H), jnp.float32),
    )(feats, w, b.reshape(1, H))


def kernel(indices, tables, proj_w, proj_b):
    F, B = indices.shape
    V, E = tables.shape[1], tables.shape[2]
    offs = (jnp.arange(F, dtype=jnp.int32) * V)[:, None]
    flat_idx = (indices + offs).T.reshape(F * B)
    flat_table = tables.reshape(F * V, E)
    feats = _gather_sc(flat_idx, flat_table).reshape(B, F * E)
    return _project_tc(feats, proj_w, proj_b)
